# Initial kernel scaffold; baseline (speedup 1.0000x reference)
import jax
import jax.numpy as jnp
from jax.experimental import pallas as pl


def kernel(x, edge_index, batch, W1, b1, W2, b2, W3, b3, L1W, L1b, L2W, L2b, L3W, L3b, CW, Cb):
    raise NotImplementedError("write your pallas kernel here")


# speedup vs baseline: 12.3716x; 12.3716x over previous
import functools
import jax
import jax.numpy as jnp
from jax import lax
from jax.experimental import pallas as pl
from jax.experimental.pallas import tpu as pltpu
from jax.experimental.pallas import tpu_sc as plsc

_N = 10000
_E = 320000

_NC = 2
_NS = 16
_NW = _NC * _NS

_NPAD = 10112
_RPT = _NPAD // _NS
_W = 32
_H = 16
_CH = 128
_PCH = 160
_NBUF = 4
_EPAD = _NS * _PCH * _CH
_EPT = _E // _NW
_FCH = _EPT // _CH
_FTAIL = _EPT - _FCH * _CH

_mesh = plsc.VectorSubcoreMesh(
    core_axis_name="c", subcore_axis_name="s",
    num_cores=_NC, num_subcores=_NS)

_sc_params = pltpu.CompilerParams(use_tc_tiling_on_sc=False,
                                  needs_layout_passes=False)

_f32 = jnp.float32


def _rsqrt16(v):
  x2 = v * 0.5
  i = plsc.bitcast(v, jnp.int32)
  i = jnp.int32(0x5F3759DF) - lax.shift_right_arithmetic(i, 1)
  y = plsc.bitcast(i, _f32)
  for _ in range(3):
    y = y * (1.5 - x2 * y * y)
  return y


def _tanh16(z):
  return 1.0 - 2.0 / (jnp.exp(2.0 * z) + 1.0)


def _ring_prop(table, src_v, dst_v, acc_sh, rows, gsem, ssem):
  for b in range(_NBUF):
    pltpu.async_copy(table.at[src_v.at[b]], rows[b], gsem[b])

  nround = _PCH // _NBUF

  def round_(k, carry):
    j0 = k * _NBUF
    for b in range(_NBUF):
      pltpu.make_async_copy(table.at[src_v.at[j0 + b]], rows[b],
                            gsem[b]).wait()
      pltpu.async_copy(rows[b], acc_sh.at[dst_v.at[j0 + b]], ssem[b],
                       add=True)
    for b in range(_NBUF):
      @pl.when(k < nround - 1)
      def _():
        pltpu.make_async_copy(rows[b], acc_sh.at[dst_v.at[j0 + b]],
                              ssem[b]).wait()
        pltpu.async_copy(table.at[src_v.at[j0 + _NBUF + b]], rows[b],
                         gsem[b])
    return carry

  lax.fori_loop(0, nround, round_, 0)
  for b in range(_NBUF):
    j = _PCH - _NBUF + b
    pltpu.make_async_copy(rows[b], acc_sh.at[dst_v.at[j]], ssem[b]).wait()


def _zero_acc(zeros_hbm, acc_sh, sid):
  pltpu.sync_copy(zeros_hbm.at[pl.ds(sid * _RPT, _RPT)],
                  acc_sh.at[pl.ds(sid * _RPT, _RPT)])


@functools.partial(
    pl.kernel,
    out_type=(jax.ShapeDtypeStruct((_NC, _NPAD, _H), _f32),
              jax.ShapeDtypeStruct((_NC, _NPAD, _H), _f32),
              jax.ShapeDtypeStruct((_NPAD, _H), _f32),
              jax.ShapeDtypeStruct((_NC, _NPAD, _H), _f32)),
    mesh=_mesh,
    scratch_types=[
        pltpu.VMEM((_PCH, _CH), jnp.int32),
        pltpu.VMEM((_PCH, _CH), jnp.int32),
        pltpu.VMEM((_CH, _H), _f32),
        pltpu.VMEM((_RPT, _H), _f32),
        pltpu.VMEM((_RPT, _H), _f32),
        pltpu.VMEM((_RPT, _H), _f32),
        pltpu.VMEM((1, _H), _f32),
        pltpu.VMEM_SHARED((_NPAD, _H), _f32),
        [pltpu.VMEM((_CH, _H), _f32)] * _NBUF,
        [pltpu.SemaphoreType.DMA] * _NBUF,
        [pltpu.SemaphoreType.DMA] * _NBUF,
        pltpu.SemaphoreType.DMA,
    ],
    compiler_params=_sc_params,
)
def _mega_kernel(h0h_hbm, srcp_hbm, dstp_hbm, ones_hbm, zeros_hbm, b1h_hbm,
                 s2_hbm, g2h_hbm, dinv_hbm, g1h_hbm,
                 src_v, dst_v, ones_v, dinv_v, wa, wb, b1_v,
                 acc_sh, rows, gsem, ssem, sem):
  cid = lax.axis_index("c")
  sid = lax.axis_index("s")
  rsl = pl.ds(sid * _RPT, _RPT)

  pltpu.sync_copy(srcp_hbm.at[sid], src_v)
  pltpu.sync_copy(dstp_hbm.at[sid], dst_v)
  pltpu.sync_copy(ones_hbm, ones_v)
  pltpu.sync_copy(b1h_hbm.at[cid], b1_v)
  _zero_acc(zeros_hbm, acc_sh, sid)
  plsc.subcore_barrier()

  def dfire(j, carry):
    pltpu.async_copy(ones_v, acc_sh.at[dst_v.at[j]], sem, add=True)
    return carry

  lax.fori_loop(0, _PCH, dfire, 0)

  def ddrain(j, carry):
    pltpu.make_async_copy(ones_v, acc_sh.at[dst_v.at[j]], sem).wait()
    return carry

  lax.fori_loop(0, _PCH, ddrain, 0)
  plsc.subcore_barrier()

  pltpu.sync_copy(acc_sh.at[rsl], dinv_v)

  def dinv_row(r, carry):
    dinv_v[r, :] = _rsqrt16(dinv_v[r, :] + 1.0)
    return carry

  lax.fori_loop(0, _RPT, dinv_row, 0)

  @pl.when(cid == 0)
  def _():
    pltpu.sync_copy(dinv_v, dinv_hbm.at[rsl])

  pltpu.sync_copy(h0h_hbm.at[cid, rsl], wa)

  def g1_row(r, carry):
    wa[r, :] = wa[r, :] * dinv_v[r, :]
    return carry

  lax.fori_loop(0, _RPT, g1_row, 0)
  pltpu.sync_copy(wa, g1h_hbm.at[cid, rsl])

  _zero_acc(zeros_hbm, acc_sh, sid)
  plsc.subcore_barrier()

  _ring_prop(g1h_hbm.at[cid], src_v, dst_v, acc_sh, rows, gsem, ssem)
  plsc.subcore_barrier()

  b1row = b1_v[0, :]
  pltpu.sync_copy(acc_sh.at[rsl], wb)

  def g2_row(r, carry):
    d = dinv_v[r, :]
    ph = d * (wb[r, :] + wa[r, :]) + b1row
    wb[r, :] = d * _tanh16(ph)
    return carry

  lax.fori_loop(0, _RPT, g2_row, 0)
  pltpu.sync_copy(wb, g2h_hbm.at[cid, rsl])

  _zero_acc(zeros_hbm, acc_sh, sid)
  plsc.subcore_barrier()

  _ring_prop(g2h_hbm.at[cid], src_v, dst_v, acc_sh, rows, gsem, ssem)
  plsc.subcore_barrier()

  pltpu.sync_copy(acc_sh.at[rsl], s2_hbm.at[cid, rsl])


@functools.partial(
    pl.kernel,
    out_type=jax.ShapeDtypeStruct((_NC, _NPAD, _H), _f32),
    mesh=_mesh,
    scratch_types=[
        pltpu.VMEM((_PCH, _CH), jnp.int32),
        pltpu.VMEM((_PCH, _CH), jnp.int32),
        pltpu.VMEM_SHARED((_NPAD, _H), _f32),
        [pltpu.VMEM((_CH, _H), _f32)] * _NBUF,
        [pltpu.SemaphoreType.DMA] * _NBUF,
        [pltpu.SemaphoreType.DMA] * _NBUF,
    ],
    compiler_params=_sc_params,
)
def _prop_kernel(gh_hbm, srcp_hbm, dstp_hbm, zeros_hbm, out_hbm,
                 src_v, dst_v, acc_sh, rows, gsem, ssem):
  cid = lax.axis_index("c")
  sid = lax.axis_index("s")
  rsl = pl.ds(sid * _RPT, _RPT)
  pltpu.sync_copy(srcp_hbm.at[sid], src_v)
  pltpu.sync_copy(dstp_hbm.at[sid], dst_v)
  _zero_acc(zeros_hbm, acc_sh, sid)
  plsc.subcore_barrier()
  _ring_prop(gh_hbm.at[cid], src_v, dst_v, acc_sh, rows, gsem, ssem)
  plsc.subcore_barrier()
  pltpu.sync_copy(acc_sh.at[rsl], out_hbm.at[cid, rsl])


@functools.partial(
    pl.kernel,
    out_type=(jax.ShapeDtypeStruct((_E, 16), _f32),
              jax.ShapeDtypeStruct((_E, 16), _f32)),
    mesh=_mesh,
    scratch_types=[
        pltpu.VMEM((_EPT,), jnp.int32),
        pltpu.VMEM((_EPT,), jnp.int32),
        [pltpu.VMEM((_CH, 16), _f32)] * 2,
        [pltpu.VMEM((_CH, 16), _f32)] * 2,
        [pltpu.SemaphoreType.DMA] * 2,
        [pltpu.SemaphoreType.DMA] * 2,
    ],
    compiler_params=_sc_params,
)
def _edge_kernel(t_hbm, src_hbm, dst_hbm, ea_hbm, eb_hbm,
                 src_v, dst_v, a_v, b_v, asem, bsem):
  wid = lax.axis_index("s") * _NC + lax.axis_index("c")
  base = wid * _EPT

  pltpu.sync_copy(src_hbm.at[pl.ds(base, _EPT)], src_v)
  pltpu.sync_copy(dst_hbm.at[pl.ds(base, _EPT)], dst_v)

  def chunk(j, b, nrows):
    row0 = base + j * _CH
    sidx = src_v.at[pl.ds(j * _CH, nrows)]
    didx = dst_v.at[pl.ds(j * _CH, nrows)]
    pltpu.async_copy(t_hbm.at[sidx], a_v[b].at[pl.ds(0, nrows)],
                     asem[b]).wait()
    pltpu.sync_copy(a_v[b].at[pl.ds(0, nrows)],
                    ea_hbm.at[pl.ds(row0, nrows)])
    pltpu.async_copy(t_hbm.at[didx], b_v[b].at[pl.ds(0, nrows)],
                     bsem[b]).wait()
    pltpu.sync_copy(b_v[b].at[pl.ds(0, nrows)],
                    eb_hbm.at[pl.ds(row0, nrows)])

  def body(j, carry):
    chunk(j, 0, _CH)
    return carry

  lax.fori_loop(0, _FCH, body, 0)
  chunk(_FCH, 1, _FTAIL)


_TCG = 8
_RBLK = _NPAD // _TCG


def _rblk(minor):
  return pl.BlockSpec((_RBLK, minor), lambda i: (i, 0))


def _rblk3(minor):
  return pl.BlockSpec((2, _RBLK, minor), lambda i: (0, i, 0))


def _full(shape):
  nd = len(shape)
  return pl.BlockSpec(shape, lambda i: (0,) * nd)


def _tc_call(body, out_shapes, in_specs, out_specs):
  return pl.pallas_call(body, grid=(_TCG,), in_specs=in_specs,
                        out_specs=out_specs, out_shape=out_shapes)


def _k1_body(x_ref, w1_ref, h0h_ref):
  h = jnp.dot(x_ref[...], w1_ref[...], preferred_element_type=_f32)
  h0h_ref[0] = h[:, 0:16]
  h0h_ref[1] = jnp.concatenate(
      [h[:, 16:18], jnp.zeros((_RBLK, 14), _f32)], axis=1)


def _k3_body(s_ref, g2h_ref, dinv_ref, w2_ref, b2_ref, g3h_ref):
  dinv = dinv_ref[:, 0:1]
  sg = s_ref[0] + g2h_ref[0]
  sg1 = s_ref[1] + g2h_ref[1]
  ph = (dinv * jnp.concatenate([sg, sg1[:, 0:2]], axis=1))
  h2 = jnp.tanh(
      jnp.dot(ph, w2_ref[...], preferred_element_type=_f32) + b2_ref[...])
  g3 = dinv * h2
  g3h_ref[0] = g3[:, 0:16]
  g3h_ref[1] = jnp.concatenate(
      [g3[:, 16:24], jnp.zeros((_RBLK, 8), _f32)], axis=1)


def _k4_body(s_ref, g3h_ref, dinv_ref, w3_ref, b3_ref, l1w_ref, l1b_ref,
             l2w_ref, l2b_ref, l3w_ref, l3b_ref, t_ref):
  dinv = dinv_ref[:, 0:1]
  sg = s_ref[0] + g3h_ref[0]
  sg1 = s_ref[1] + g3h_ref[1]
  ph = dinv * jnp.concatenate([sg, sg1[:, 0:8]], axis=1)
  t3 = jnp.tanh(
      jnp.dot(ph, w3_ref[...], preferred_element_type=_f32) + b3_ref[...])
  t4 = jnp.tanh(
      jnp.dot(t3, l1w_ref[...], preferred_element_type=_f32) + l1b_ref[...])
  t5 = jnp.tanh(
      jnp.dot(t4, l2w_ref[...], preferred_element_type=_f32) + l2b_ref[...])
  h = jnp.tanh(
      jnp.dot(t5, l3w_ref[...], preferred_element_type=_f32) + l3b_ref[...])
  t_ref[...] = jnp.concatenate([h, jnp.zeros((_RBLK, 4), _f32)], axis=1)


def _k5_body(ea_ref, eb_ref, cw_ref, cb_ref, e_ref, o_ref):
  e = jnp.concatenate([ea_ref[:, 0:12], eb_ref[:, 0:12]], axis=1)
  e_ref[...] = e
  o_ref[...] = (
      jnp.dot(e, cw_ref[...], preferred_element_type=_f32) + cb_ref[...])


_K5_BLK = 4000


def kernel(x, edge_index, batch, W1, b1, W2, b2, W3, b3,
           L1W, L1b, L2W, L2b, L3W, L3b, CW, Cb):
  del batch
  src = edge_index[0].astype(jnp.int32)
  dst = edge_index[1].astype(jnp.int32)

  sent = jnp.full((_EPAD - _E,), _N, jnp.int32)
  srcp = jnp.concatenate([src, sent]).reshape(_NS, _PCH, _CH)
  dstp = jnp.concatenate([dst, sent]).reshape(_NS, _PCH, _CH)

  xpad = jnp.zeros((_NPAD, 128), _f32).at[:_N].set(x)
  ones16 = jnp.ones((_CH, _H), _f32)
  zeros16 = jnp.zeros((_NPAD, _H), _f32)
  b1h = jnp.zeros((_NC, 1, _H), _f32).at[0, 0, :16].set(b1[:16]) \
      .at[1, 0, :2].set(b1[16:18])

  h0h = _tc_call(
      _k1_body, jax.ShapeDtypeStruct((_NC, _NPAD, _H), _f32),
      [_rblk(128), _full((128, 18))], _rblk3(_H))(xpad, W1)

  s2, g2h, dinv16, _g1h = _mega_kernel(h0h, srcp, dstp, ones16, zeros16,
                                       b1h)

  g3h = _tc_call(
      _k3_body, jax.ShapeDtypeStruct((_NC, _NPAD, _H), _f32),
      [_rblk3(_H), _rblk3(_H), _rblk(_H), _full((18, 24)), _full((1, 24))],
      _rblk3(_H))(s2, g2h, dinv16, W2, b2.reshape(1, 24))

  s3 = _prop_kernel(g3h, srcp, dstp, zeros16)

  t = _tc_call(
      _k4_body, jax.ShapeDtypeStruct((_NPAD, 16), _f32),
      [_rblk3(_H), _rblk3(_H), _rblk(_H), _full((24, 256)), _full((1, 256)),
       _full((256, 24)), _full((1, 24)), _full((24, 18)), _full((1, 18)),
       _full((18, 12)), _full((1, 12))],
      _rblk(16))(s3, g3h, dinv16, W3, b3.reshape(1, 256), L1W,
                 L1b.reshape(1, 24), L2W, L2b.reshape(1, 18), L3W,
                 L3b.reshape(1, 12))

  ea, eb = _edge_kernel(t, src, dst)

  e, out = pl.pallas_call(
      _k5_body,
      grid=(_E // _K5_BLK,),
      in_specs=[
          pl.BlockSpec((_K5_BLK, 16), lambda i: (i, 0)),
          pl.BlockSpec((_K5_BLK, 16), lambda i: (i, 0)),
          pl.BlockSpec((24, 10), lambda i: (0, 0)),
          pl.BlockSpec((1, 10), lambda i: (0, 0)),
      ],
      out_specs=[
          pl.BlockSpec((_K5_BLK, 24), lambda i: (i, 0)),
          pl.BlockSpec((_K5_BLK, 10), lambda i: (i, 0)),
      ],
      out_shape=[
          jax.ShapeDtypeStruct((_E, 24), _f32),
          jax.ShapeDtypeStruct((_E, 10), _f32),
      ],
  )(ea, eb, CW, Cb.reshape(1, 10))

  return (out, e)


# speedup vs baseline: 15.3847x; 1.2435x over previous
import functools
import jax
import jax.numpy as jnp
from jax import lax
from jax.experimental import pallas as pl
from jax.experimental.pallas import tpu as pltpu
from jax.experimental.pallas import tpu_sc as plsc

_N = 10000
_E = 320000

_NC = 2
_NS = 16
_NW = _NC * _NS

_NPAD = 10112
_RPT = _NPAD // _NS
_W = 32
_H = 16
_CH = 128
_PCH = 160
_NBUF = 4
_EPAD = _NS * _PCH * _CH
_EPT = _E // _NW
_FCH = _EPT // _CH
_FTAIL = _EPT - _FCH * _CH

_mesh = plsc.VectorSubcoreMesh(
    core_axis_name="c", subcore_axis_name="s",
    num_cores=_NC, num_subcores=_NS)

_sc_params = pltpu.CompilerParams(use_tc_tiling_on_sc=False,
                                  needs_layout_passes=False)

_f32 = jnp.float32


def _rsqrt16(v):
  x2 = v * 0.5
  i = plsc.bitcast(v, jnp.int32)
  i = jnp.int32(0x5F3759DF) - lax.shift_right_arithmetic(i, 1)
  y = plsc.bitcast(i, _f32)
  for _ in range(3):
    y = y * (1.5 - x2 * y * y)
  return y


def _tanh16(z):
  return 1.0 - 2.0 / (jnp.exp(2.0 * z) + 1.0)


def _ring_prop(table, src_v, dst_v, acc_sh, rows, gsem, ssem):
  for b in range(_NBUF):
    pltpu.async_copy(table.at[src_v.at[b]], rows[b], gsem[b])

  nround = _PCH // _NBUF

  def round_(k, carry):
    j0 = k * _NBUF
    for b in range(_NBUF):
      pltpu.make_async_copy(table.at[src_v.at[j0 + b]], rows[b],
                            gsem[b]).wait()
      pltpu.async_copy(rows[b], acc_sh.at[dst_v.at[j0 + b]], ssem[b],
                       add=True)
    for b in range(_NBUF):
      @pl.when(k < nround - 1)
      def _():
        pltpu.make_async_copy(rows[b], acc_sh.at[dst_v.at[j0 + b]],
                              ssem[b]).wait()
        pltpu.async_copy(table.at[src_v.at[j0 + _NBUF + b]], rows[b],
                         gsem[b])
    return carry

  lax.fori_loop(0, nround, round_, 0)
  for b in range(_NBUF):
    j = _PCH - _NBUF + b
    pltpu.make_async_copy(rows[b], acc_sh.at[dst_v.at[j]], ssem[b]).wait()


def _zero_acc(zeros_hbm, acc_sh, sid):
  pltpu.sync_copy(zeros_hbm.at[pl.ds(sid * _RPT, _RPT)],
                  acc_sh.at[pl.ds(sid * _RPT, _RPT)])


@functools.partial(
    pl.kernel,
    out_type=(jax.ShapeDtypeStruct((_NC, _NPAD, _H), _f32),
              jax.ShapeDtypeStruct((_NC, _NPAD, _H), _f32),
              jax.ShapeDtypeStruct((_NPAD, _H), _f32),
              jax.ShapeDtypeStruct((_NC, _NPAD, _H), _f32)),
    mesh=_mesh,
    scratch_types=[
        pltpu.VMEM((_PCH, _CH), jnp.int32),
        pltpu.VMEM((_PCH, _CH), jnp.int32),
        pltpu.VMEM((_CH, _H), _f32),
        pltpu.VMEM((_RPT, _H), _f32),
        pltpu.VMEM((_RPT, _H), _f32),
        pltpu.VMEM((_RPT, _H), _f32),
        pltpu.VMEM((1, _H), _f32),
        pltpu.VMEM_SHARED((_NPAD, _H), _f32),
        [pltpu.VMEM((_CH, _H), _f32)] * _NBUF,
        [pltpu.SemaphoreType.DMA] * _NBUF,
        [pltpu.SemaphoreType.DMA] * _NBUF,
        pltpu.SemaphoreType.DMA,
    ],
    compiler_params=_sc_params,
)
def _mega_kernel(h0h_hbm, srcp_hbm, dstp_hbm, ones_hbm, zeros_hbm, b1h_hbm,
                 s2_hbm, g2h_hbm, dinv_hbm, g1h_hbm,
                 src_v, dst_v, ones_v, dinv_v, wa, wb, b1_v,
                 acc_sh, rows, gsem, ssem, sem):
  cid = lax.axis_index("c")
  sid = lax.axis_index("s")
  rsl = pl.ds(sid * _RPT, _RPT)

  pltpu.sync_copy(srcp_hbm.at[sid], src_v)
  pltpu.sync_copy(dstp_hbm.at[sid], dst_v)
  pltpu.sync_copy(ones_hbm, ones_v)
  pltpu.sync_copy(b1h_hbm.at[cid], b1_v)
  _zero_acc(zeros_hbm, acc_sh, sid)
  plsc.subcore_barrier()

  def dfire(j, carry):
    pltpu.async_copy(ones_v, acc_sh.at[dst_v.at[j]], sem, add=True)
    return carry

  lax.fori_loop(0, _PCH, dfire, 0)

  def ddrain(j, carry):
    pltpu.make_async_copy(ones_v, acc_sh.at[dst_v.at[j]], sem).wait()
    return carry

  lax.fori_loop(0, _PCH, ddrain, 0)
  plsc.subcore_barrier()

  pltpu.sync_copy(acc_sh.at[rsl], dinv_v)

  def dinv_row(r, carry):
    dinv_v[r, :] = _rsqrt16(dinv_v[r, :] + 1.0)
    return carry

  lax.fori_loop(0, _RPT, dinv_row, 0)

  @pl.when(cid == 0)
  def _():
    pltpu.sync_copy(dinv_v, dinv_hbm.at[rsl])

  pltpu.sync_copy(h0h_hbm.at[cid, rsl], wa)

  def g1_row(r, carry):
    wa[r, :] = wa[r, :] * dinv_v[r, :]
    return carry

  lax.fori_loop(0, _RPT, g1_row, 0)
  pltpu.sync_copy(wa, g1h_hbm.at[cid, rsl])

  _zero_acc(zeros_hbm, acc_sh, sid)
  plsc.subcore_barrier()

  _ring_prop(g1h_hbm.at[cid], src_v, dst_v, acc_sh, rows, gsem, ssem)
  plsc.subcore_barrier()

  b1row = b1_v[0, :]
  pltpu.sync_copy(acc_sh.at[rsl], wb)

  def g2_row(r, carry):
    d = dinv_v[r, :]
    ph = d * (wb[r, :] + wa[r, :]) + b1row
    wb[r, :] = d * _tanh16(ph)
    return carry

  lax.fori_loop(0, _RPT, g2_row, 0)
  pltpu.sync_copy(wb, g2h_hbm.at[cid, rsl])

  _zero_acc(zeros_hbm, acc_sh, sid)
  plsc.subcore_barrier()

  _ring_prop(g2h_hbm.at[cid], src_v, dst_v, acc_sh, rows, gsem, ssem)
  plsc.subcore_barrier()

  pltpu.sync_copy(acc_sh.at[rsl], s2_hbm.at[cid, rsl])


@functools.partial(
    pl.kernel,
    out_type=jax.ShapeDtypeStruct((_NC, _NPAD, _H), _f32),
    mesh=_mesh,
    scratch_types=[
        pltpu.VMEM((_PCH, _CH), jnp.int32),
        pltpu.VMEM((_PCH, _CH), jnp.int32),
        pltpu.VMEM_SHARED((_NPAD, _H), _f32),
        [pltpu.VMEM((_CH, _H), _f32)] * _NBUF,
        [pltpu.SemaphoreType.DMA] * _NBUF,
        [pltpu.SemaphoreType.DMA] * _NBUF,
    ],
    compiler_params=_sc_params,
)
def _prop_kernel(gh_hbm, srcp_hbm, dstp_hbm, zeros_hbm, out_hbm,
                 src_v, dst_v, acc_sh, rows, gsem, ssem):
  cid = lax.axis_index("c")
  sid = lax.axis_index("s")
  rsl = pl.ds(sid * _RPT, _RPT)
  pltpu.sync_copy(srcp_hbm.at[sid], src_v)
  pltpu.sync_copy(dstp_hbm.at[sid], dst_v)
  _zero_acc(zeros_hbm, acc_sh, sid)
  plsc.subcore_barrier()
  _ring_prop(gh_hbm.at[cid], src_v, dst_v, acc_sh, rows, gsem, ssem)
  plsc.subcore_barrier()
  pltpu.sync_copy(acc_sh.at[rsl], out_hbm.at[cid, rsl])


@functools.partial(
    pl.kernel,
    out_type=(jax.ShapeDtypeStruct((_E, 16), _f32),
              jax.ShapeDtypeStruct((_E, 16), _f32)),
    mesh=_mesh,
    scratch_types=[
        pltpu.VMEM((_EPT,), jnp.int32),
        pltpu.VMEM((_EPT,), jnp.int32),
        [pltpu.VMEM((_CH, 16), _f32)] * 2,
        [pltpu.VMEM((_CH, 16), _f32)] * 2,
        [pltpu.SemaphoreType.DMA] * 2,
        [pltpu.SemaphoreType.DMA] * 2,
    ],
    compiler_params=_sc_params,
)
def _edge_kernel(t_hbm, src_hbm, dst_hbm, ea_hbm, eb_hbm,
                 src_v, dst_v, a_v, b_v, asem, bsem):
  wid = lax.axis_index("s") * _NC + lax.axis_index("c")
  base = wid * _EPT

  pltpu.sync_copy(src_hbm.at[pl.ds(base, _EPT)], src_v)
  pltpu.sync_copy(dst_hbm.at[pl.ds(base, _EPT)], dst_v)

  def chunk(j, b, nrows):
    row0 = base + j * _CH
    sidx = src_v.at[pl.ds(j * _CH, nrows)]
    didx = dst_v.at[pl.ds(j * _CH, nrows)]
    pltpu.async_copy(t_hbm.at[sidx], a_v[b].at[pl.ds(0, nrows)],
                     asem[b]).wait()
    pltpu.sync_copy(a_v[b].at[pl.ds(0, nrows)],
                    ea_hbm.at[pl.ds(row0, nrows)])
    pltpu.async_copy(t_hbm.at[didx], b_v[b].at[pl.ds(0, nrows)],
                     bsem[b]).wait()
    pltpu.sync_copy(b_v[b].at[pl.ds(0, nrows)],
                    eb_hbm.at[pl.ds(row0, nrows)])

  def body(j, carry):
    chunk(j, 0, _CH)
    return carry

  lax.fori_loop(0, _FCH, body, 0)
  chunk(_FCH, 1, _FTAIL)


_TCG = 8
_RBLK = _NPAD // _TCG


def _rblk(minor):
  return pl.BlockSpec((_RBLK, minor), lambda i: (i, 0))


def _rblk3(minor):
  return pl.BlockSpec((2, _RBLK, minor), lambda i: (0, i, 0))


def _full(shape):
  nd = len(shape)
  return pl.BlockSpec(shape, lambda i: (0,) * nd)


def _tc_call(body, out_shapes, in_specs, out_specs):
  return pl.pallas_call(body, grid=(_TCG,), in_specs=in_specs,
                        out_specs=out_specs, out_shape=out_shapes)


def _k1_body(x_ref, w1_ref, h0h_ref):
  h = jnp.dot(x_ref[...], w1_ref[...], preferred_element_type=_f32)
  h0h_ref[0] = h[:, 0:16]
  h0h_ref[1] = jnp.concatenate(
      [h[:, 16:18], jnp.zeros((_RBLK, 14), _f32)], axis=1)


def _k3_body(s_ref, g2h_ref, dinv_ref, w2_ref, b2_ref, g3h_ref):
  dinv = dinv_ref[:, 0:1]
  sg = s_ref[0] + g2h_ref[0]
  sg1 = s_ref[1] + g2h_ref[1]
  ph = (dinv * jnp.concatenate([sg, sg1[:, 0:2]], axis=1))
  h2 = jnp.tanh(
      jnp.dot(ph, w2_ref[...], preferred_element_type=_f32) + b2_ref[...])
  g3 = dinv * h2
  g3h_ref[0] = g3[:, 0:16]
  g3h_ref[1] = jnp.concatenate(
      [g3[:, 16:24], jnp.zeros((_RBLK, 8), _f32)], axis=1)


def _k4_body(s_ref, g3h_ref, dinv_ref, w3_ref, b3_ref, l1w_ref, l1b_ref,
             l2w_ref, l2b_ref, l3w_ref, l3b_ref, t_ref):
  dinv = dinv_ref[:, 0:1]
  sg = s_ref[0] + g3h_ref[0]
  sg1 = s_ref[1] + g3h_ref[1]
  ph = dinv * jnp.concatenate([sg, sg1[:, 0:8]], axis=1)
  t3 = jnp.tanh(
      jnp.dot(ph, w3_ref[...], preferred_element_type=_f32) + b3_ref[...])
  t4 = jnp.tanh(
      jnp.dot(t3, l1w_ref[...], preferred_element_type=_f32) + l1b_ref[...])
  t5 = jnp.tanh(
      jnp.dot(t4, l2w_ref[...], preferred_element_type=_f32) + l2b_ref[...])
  h = jnp.tanh(
      jnp.dot(t5, l3w_ref[...], preferred_element_type=_f32) + l3b_ref[...])
  t_ref[...] = jnp.concatenate([h, jnp.zeros((_RBLK, 4), _f32)], axis=1)


def _k5_body(ea_ref, eb_ref, cw_ref, cb_ref, et_ref, ot_ref):
  e = jnp.concatenate([ea_ref[:, 0:12], eb_ref[:, 0:12]], axis=1)
  et_ref[...] = e.T
  o = jnp.dot(e, cw_ref[...], preferred_element_type=_f32) + cb_ref[...]
  ot_ref[...] = o.T


_K5_BLK = 6400
_K5_R8 = _K5_BLK * 16 // 128


def kernel(x, edge_index, batch, W1, b1, W2, b2, W3, b3,
           L1W, L1b, L2W, L2b, L3W, L3b, CW, Cb):
  del batch
  src = edge_index[0].astype(jnp.int32)
  dst = edge_index[1].astype(jnp.int32)

  sent = jnp.full((_EPAD - _E,), _N, jnp.int32)
  srcp = jnp.concatenate([src, sent]).reshape(_NS, _PCH, _CH)
  dstp = jnp.concatenate([dst, sent]).reshape(_NS, _PCH, _CH)

  xpad = jnp.zeros((_NPAD, 128), _f32).at[:_N].set(x)
  ones16 = jnp.ones((_CH, _H), _f32)
  zeros16 = jnp.zeros((_NPAD, _H), _f32)
  b1h = jnp.zeros((_NC, 1, _H), _f32).at[0, 0, :16].set(b1[:16]) \
      .at[1, 0, :2].set(b1[16:18])

  h0h = _tc_call(
      _k1_body, jax.ShapeDtypeStruct((_NC, _NPAD, _H), _f32),
      [_rblk(128), _full((128, 18))], _rblk3(_H))(xpad, W1)

  s2, g2h, dinv16, _g1h = _mega_kernel(h0h, srcp, dstp, ones16, zeros16,
                                       b1h)

  g3h = _tc_call(
      _k3_body, jax.ShapeDtypeStruct((_NC, _NPAD, _H), _f32),
      [_rblk3(_H), _rblk3(_H), _rblk(_H), _full((18, 24)), _full((1, 24))],
      _rblk3(_H))(s2, g2h, dinv16, W2, b2.reshape(1, 24))

  s3 = _prop_kernel(g3h, srcp, dstp, zeros16)

  t = _tc_call(
      _k4_body, jax.ShapeDtypeStruct((_NPAD, 16), _f32),
      [_rblk3(_H), _rblk3(_H), _rblk(_H), _full((24, 256)), _full((1, 256)),
       _full((256, 24)), _full((1, 24)), _full((24, 18)), _full((1, 18)),
       _full((18, 12)), _full((1, 12))],
      _rblk(16))(s3, g3h, dinv16, W3, b3.reshape(1, 256), L1W,
                 L1b.reshape(1, 24), L2W, L2b.reshape(1, 18), L3W,
                 L3b.reshape(1, 12))

  ea, eb = _edge_kernel(t, src, dst)

  et, ot = pl.pallas_call(
      _k5_body,
      grid=(_E // _K5_BLK,),
      in_specs=[
          pl.BlockSpec((_K5_BLK, 16), lambda i: (i, 0)),
          pl.BlockSpec((_K5_BLK, 16), lambda i: (i, 0)),
          pl.BlockSpec((24, 10), lambda i: (0, 0)),
          pl.BlockSpec((1, 10), lambda i: (0, 0)),
      ],
      out_specs=[
          pl.BlockSpec((24, _K5_BLK), lambda i: (0, i)),
          pl.BlockSpec((10, _K5_BLK), lambda i: (0, i)),
      ],
      out_shape=[
          jax.ShapeDtypeStruct((24, _E), _f32),
          jax.ShapeDtypeStruct((10, _E), _f32),
      ],
  )(ea, eb, CW, Cb.reshape(1, 10))

  return (ot.T, et.T)


# speedup vs baseline: 17.0069x; 1.1054x over previous
import functools
import jax
import jax.numpy as jnp
from jax import lax
from jax.experimental import pallas as pl
from jax.experimental.pallas import tpu as pltpu
from jax.experimental.pallas import tpu_sc as plsc

_N = 10000
_E = 320000

_NC = 2
_NS = 16
_NW = _NC * _NS

_NPAD = 10112
_RPT = _NPAD // _NS
_W = 32
_H = 16
_CH = 128
_PCH = 160
_NBUF = 4
_EPAD = _NS * _PCH * _CH
_EPT = _E // _NW
_FCH = _EPT // _CH
_FTAIL = _EPT - _FCH * _CH

_mesh = plsc.VectorSubcoreMesh(
    core_axis_name="c", subcore_axis_name="s",
    num_cores=_NC, num_subcores=_NS)

_sc_params = pltpu.CompilerParams(use_tc_tiling_on_sc=False,
                                  needs_layout_passes=False)

_f32 = jnp.float32


def _rsqrt16(v):
  x2 = v * 0.5
  i = plsc.bitcast(v, jnp.int32)
  i = jnp.int32(0x5F3759DF) - lax.shift_right_arithmetic(i, 1)
  y = plsc.bitcast(i, _f32)
  for _ in range(3):
    y = y * (1.5 - x2 * y * y)
  return y


def _tanh16(z):
  return 1.0 - 2.0 / (jnp.exp(2.0 * z) + 1.0)


def _ring_prop(table, src_v, dst_v, acc_sh, rows, gsem, ssem):
  for b in range(_NBUF):
    pltpu.async_copy(table.at[src_v.at[b]], rows[b], gsem[b])

  nround = _PCH // _NBUF

  def round_(k, carry):
    j0 = k * _NBUF
    for b in range(_NBUF):
      pltpu.make_async_copy(table.at[src_v.at[j0 + b]], rows[b],
                            gsem[b]).wait()
      pltpu.async_copy(rows[b], acc_sh.at[dst_v.at[j0 + b]], ssem[b],
                       add=True)
    for b in range(_NBUF):
      @pl.when(k < nround - 1)
      def _():
        pltpu.make_async_copy(rows[b], acc_sh.at[dst_v.at[j0 + b]],
                              ssem[b]).wait()
        pltpu.async_copy(table.at[src_v.at[j0 + _NBUF + b]], rows[b],
                         gsem[b])
    return carry

  lax.fori_loop(0, nround, round_, 0)
  for b in range(_NBUF):
    j = _PCH - _NBUF + b
    pltpu.make_async_copy(rows[b], acc_sh.at[dst_v.at[j]], ssem[b]).wait()


def _zero_acc(zeros_hbm, acc_sh, sid):
  pltpu.sync_copy(zeros_hbm.at[pl.ds(sid * _RPT, _RPT)],
                  acc_sh.at[pl.ds(sid * _RPT, _RPT)])


@functools.partial(
    pl.kernel,
    out_type=(jax.ShapeDtypeStruct((_NC, _NPAD, _H), _f32),
              jax.ShapeDtypeStruct((_NC, _NPAD, _H), _f32),
              jax.ShapeDtypeStruct((_NPAD, _H), _f32),
              jax.ShapeDtypeStruct((_NC, _NPAD, _H), _f32)),
    mesh=_mesh,
    scratch_types=[
        pltpu.VMEM((_PCH, _CH), jnp.int32),
        pltpu.VMEM((_PCH, _CH), jnp.int32),
        pltpu.VMEM((_CH, _H), _f32),
        pltpu.VMEM((_RPT, _H), _f32),
        pltpu.VMEM((_RPT, _H), _f32),
        pltpu.VMEM((_RPT, _H), _f32),
        pltpu.VMEM((1, _H), _f32),
        pltpu.VMEM_SHARED((_NPAD, _H), _f32),
        [pltpu.VMEM((_CH, _H), _f32)] * _NBUF,
        [pltpu.SemaphoreType.DMA] * _NBUF,
        [pltpu.SemaphoreType.DMA] * _NBUF,
        pltpu.SemaphoreType.DMA,
    ],
    compiler_params=_sc_params,
)
def _mega_kernel(h0h_hbm, srcp_hbm, dstp_hbm, ones_hbm, zeros_hbm, b1h_hbm,
                 s2_hbm, g2h_hbm, dinv_hbm, g1h_hbm,
                 src_v, dst_v, ones_v, dinv_v, wa, wb, b1_v,
                 acc_sh, rows, gsem, ssem, sem):
  cid = lax.axis_index("c")
  sid = lax.axis_index("s")
  rsl = pl.ds(sid * _RPT, _RPT)

  pltpu.sync_copy(srcp_hbm.at[sid], src_v)
  pltpu.sync_copy(dstp_hbm.at[sid], dst_v)
  pltpu.sync_copy(ones_hbm, ones_v)
  pltpu.sync_copy(b1h_hbm.at[cid], b1_v)
  _zero_acc(zeros_hbm, acc_sh, sid)
  plsc.subcore_barrier()

  def dfire(j, carry):
    pltpu.async_copy(ones_v, acc_sh.at[dst_v.at[j]], sem, add=True)
    return carry

  lax.fori_loop(0, _PCH, dfire, 0)

  def ddrain(j, carry):
    pltpu.make_async_copy(ones_v, acc_sh.at[dst_v.at[j]], sem).wait()
    return carry

  lax.fori_loop(0, _PCH, ddrain, 0)
  plsc.subcore_barrier()

  pltpu.sync_copy(acc_sh.at[rsl], dinv_v)

  def dinv_row(r, carry):
    dinv_v[r, :] = _rsqrt16(dinv_v[r, :] + 1.0)
    return carry

  lax.fori_loop(0, _RPT, dinv_row, 0)

  @pl.when(cid == 0)
  def _():
    pltpu.sync_copy(dinv_v, dinv_hbm.at[rsl])

  pltpu.sync_copy(h0h_hbm.at[cid, rsl], wa)

  def g1_row(r, carry):
    wa[r, :] = wa[r, :] * dinv_v[r, :]
    return carry

  lax.fori_loop(0, _RPT, g1_row, 0)
  pltpu.sync_copy(wa, g1h_hbm.at[cid, rsl])

  _zero_acc(zeros_hbm, acc_sh, sid)
  plsc.subcore_barrier()

  _ring_prop(g1h_hbm.at[cid], src_v, dst_v, acc_sh, rows, gsem, ssem)
  plsc.subcore_barrier()

  b1row = b1_v[0, :]
  pltpu.sync_copy(acc_sh.at[rsl], wb)

  def g2_row(r, carry):
    d = dinv_v[r, :]
    ph = d * (wb[r, :] + wa[r, :]) + b1row
    wb[r, :] = d * _tanh16(ph)
    return carry

  lax.fori_loop(0, _RPT, g2_row, 0)
  pltpu.sync_copy(wb, g2h_hbm.at[cid, rsl])

  _zero_acc(zeros_hbm, acc_sh, sid)
  plsc.subcore_barrier()

  _ring_prop(g2h_hbm.at[cid], src_v, dst_v, acc_sh, rows, gsem, ssem)
  plsc.subcore_barrier()

  pltpu.sync_copy(acc_sh.at[rsl], s2_hbm.at[cid, rsl])


@functools.partial(
    pl.kernel,
    out_type=jax.ShapeDtypeStruct((_NC, _NPAD, _H), _f32),
    mesh=_mesh,
    scratch_types=[
        pltpu.VMEM((_PCH, _CH), jnp.int32),
        pltpu.VMEM((_PCH, _CH), jnp.int32),
        pltpu.VMEM_SHARED((_NPAD, _H), _f32),
        [pltpu.VMEM((_CH, _H), _f32)] * _NBUF,
        [pltpu.SemaphoreType.DMA] * _NBUF,
        [pltpu.SemaphoreType.DMA] * _NBUF,
    ],
    compiler_params=_sc_params,
)
def _prop_kernel(gh_hbm, srcp_hbm, dstp_hbm, zeros_hbm, out_hbm,
                 src_v, dst_v, acc_sh, rows, gsem, ssem):
  cid = lax.axis_index("c")
  sid = lax.axis_index("s")
  rsl = pl.ds(sid * _RPT, _RPT)
  pltpu.sync_copy(srcp_hbm.at[sid], src_v)
  pltpu.sync_copy(dstp_hbm.at[sid], dst_v)
  _zero_acc(zeros_hbm, acc_sh, sid)
  plsc.subcore_barrier()
  _ring_prop(gh_hbm.at[cid], src_v, dst_v, acc_sh, rows, gsem, ssem)
  plsc.subcore_barrier()
  pltpu.sync_copy(acc_sh.at[rsl], out_hbm.at[cid, rsl])


@functools.partial(
    pl.kernel,
    out_type=jax.ShapeDtypeStruct((_E, 32), _f32),
    mesh=_mesh,
    scratch_types=[
        pltpu.VMEM((_EPT,), jnp.int32),
        pltpu.VMEM((_EPT,), jnp.int32),
        [pltpu.VMEM((_CH, 16), _f32)] * 2,
        [pltpu.VMEM((_CH, 16), _f32)] * 2,
        [pltpu.SemaphoreType.DMA] * 2,
        [pltpu.SemaphoreType.DMA] * 2,
    ],
    compiler_params=_sc_params,
)
def _edge_kernel(t_hbm, src_hbm, dst_hbm, ee_hbm,
                 src_v, dst_v, a_v, b_v, asem, bsem):
  wid = lax.axis_index("s") * _NC + lax.axis_index("c")
  base = wid * _EPT

  pltpu.sync_copy(src_hbm.at[pl.ds(base, _EPT)], src_v)
  pltpu.sync_copy(dst_hbm.at[pl.ds(base, _EPT)], dst_v)

  def chunk(j, b, nrows):
    row0 = base + j * _CH
    sidx = src_v.at[pl.ds(j * _CH, nrows)]
    didx = dst_v.at[pl.ds(j * _CH, nrows)]
    pltpu.async_copy(t_hbm.at[sidx], a_v[b].at[pl.ds(0, nrows)],
                     asem[b]).wait()
    pltpu.sync_copy(a_v[b].at[pl.ds(0, nrows)],
                    ee_hbm.at[pl.ds(row0, nrows), pl.ds(0, 16)])
    pltpu.async_copy(t_hbm.at[didx], b_v[b].at[pl.ds(0, nrows)],
                     bsem[b]).wait()
    pltpu.sync_copy(b_v[b].at[pl.ds(0, nrows)],
                    ee_hbm.at[pl.ds(row0, nrows), pl.ds(16, 16)])

  def body(j, carry):
    chunk(j, 0, _CH)
    return carry

  lax.fori_loop(0, _FCH, body, 0)
  chunk(_FCH, 1, _FTAIL)


_TCG = 8
_RBLK = _NPAD // _TCG


def _rblk(minor):
  return pl.BlockSpec((_RBLK, minor), lambda i: (i, 0))


def _rblk3(minor):
  return pl.BlockSpec((2, _RBLK, minor), lambda i: (0, i, 0))


def _full(shape):
  nd = len(shape)
  return pl.BlockSpec(shape, lambda i: (0,) * nd)


def _tc_call(body, out_shapes, in_specs, out_specs):
  return pl.pallas_call(body, grid=(_TCG,), in_specs=in_specs,
                        out_specs=out_specs, out_shape=out_shapes)


def _k1_body(x_ref, w1_ref, h0h_ref):
  h = jnp.dot(x_ref[...], w1_ref[...], preferred_element_type=_f32)
  h0h_ref[0] = h[:, 0:16]
  h0h_ref[1] = jnp.concatenate(
      [h[:, 16:18], jnp.zeros((_RBLK, 14), _f32)], axis=1)


def _k3_body(s_ref, g2h_ref, dinv_ref, w2_ref, b2_ref, g3h_ref):
  dinv = dinv_ref[:, 0:1]
  sg = s_ref[0] + g2h_ref[0]
  sg1 = s_ref[1] + g2h_ref[1]
  ph = (dinv * jnp.concatenate([sg, sg1[:, 0:2]], axis=1))
  h2 = jnp.tanh(
      jnp.dot(ph, w2_ref[...], preferred_element_type=_f32) + b2_ref[...])
  g3 = dinv * h2
  g3h_ref[0] = g3[:, 0:16]
  g3h_ref[1] = jnp.concatenate(
      [g3[:, 16:24], jnp.zeros((_RBLK, 8), _f32)], axis=1)


def _k4_body(s_ref, g3h_ref, dinv_ref, w3_ref, b3_ref, l1w_ref, l1b_ref,
             l2w_ref, l2b_ref, l3w_ref, l3b_ref, t_ref):
  dinv = dinv_ref[:, 0:1]
  sg = s_ref[0] + g3h_ref[0]
  sg1 = s_ref[1] + g3h_ref[1]
  ph = dinv * jnp.concatenate([sg, sg1[:, 0:8]], axis=1)
  t3 = jnp.tanh(
      jnp.dot(ph, w3_ref[...], preferred_element_type=_f32) + b3_ref[...])
  t4 = jnp.tanh(
      jnp.dot(t3, l1w_ref[...], preferred_element_type=_f32) + l1b_ref[...])
  t5 = jnp.tanh(
      jnp.dot(t4, l2w_ref[...], preferred_element_type=_f32) + l2b_ref[...])
  h = jnp.tanh(
      jnp.dot(t5, l3w_ref[...], preferred_element_type=_f32) + l3b_ref[...])
  t_ref[...] = jnp.concatenate([h, jnp.zeros((_RBLK, 4), _f32)], axis=1)


def _k5_body(ee_ref, cw_ref, cb_ref, et_ref, ot_ref):
  e = jnp.concatenate([ee_ref[:, 0:12], ee_ref[:, 16:28]], axis=1)
  et_ref[...] = e.T
  o = jnp.dot(e, cw_ref[...], preferred_element_type=_f32) + cb_ref[...]
  ot_ref[...] = o.T


_K5_BLK = 6400
_K5_R8 = _K5_BLK * 16 // 128


def kernel(x, edge_index, batch, W1, b1, W2, b2, W3, b3,
           L1W, L1b, L2W, L2b, L3W, L3b, CW, Cb):
  del batch
  src = edge_index[0].astype(jnp.int32)
  dst = edge_index[1].astype(jnp.int32)

  sent = jnp.full((_EPAD - _E,), _N, jnp.int32)
  srcp = jnp.concatenate([src, sent]).reshape(_NS, _PCH, _CH)
  dstp = jnp.concatenate([dst, sent]).reshape(_NS, _PCH, _CH)

  xpad = jnp.zeros((_NPAD, 128), _f32).at[:_N].set(x)
  ones16 = jnp.ones((_CH, _H), _f32)
  zeros16 = jnp.zeros((_NPAD, _H), _f32)
  b1h = jnp.zeros((_NC, 1, _H), _f32).at[0, 0, :16].set(b1[:16]) \
      .at[1, 0, :2].set(b1[16:18])

  h0h = _tc_call(
      _k1_body, jax.ShapeDtypeStruct((_NC, _NPAD, _H), _f32),
      [_rblk(128), _full((128, 18))], _rblk3(_H))(xpad, W1)

  s2, g2h, dinv16, _g1h = _mega_kernel(h0h, srcp, dstp, ones16, zeros16,
                                       b1h)

  g3h = _tc_call(
      _k3_body, jax.ShapeDtypeStruct((_NC, _NPAD, _H), _f32),
      [_rblk3(_H), _rblk3(_H), _rblk(_H), _full((18, 24)), _full((1, 24))],
      _rblk3(_H))(s2, g2h, dinv16, W2, b2.reshape(1, 24))

  s3 = _prop_kernel(g3h, srcp, dstp, zeros16)

  t = _tc_call(
      _k4_body, jax.ShapeDtypeStruct((_NPAD, 16), _f32),
      [_rblk3(_H), _rblk3(_H), _rblk(_H), _full((24, 256)), _full((1, 256)),
       _full((256, 24)), _full((1, 24)), _full((24, 18)), _full((1, 18)),
       _full((18, 12)), _full((1, 12))],
      _rblk(16))(s3, g3h, dinv16, W3, b3.reshape(1, 256), L1W,
                 L1b.reshape(1, 24), L2W, L2b.reshape(1, 18), L3W,
                 L3b.reshape(1, 12))

  ee = _edge_kernel(t, src, dst)

  et, ot = pl.pallas_call(
      _k5_body,
      grid=(_E // _K5_BLK,),
      in_specs=[
          pl.BlockSpec((_K5_BLK, 32), lambda i: (i, 0)),
          pl.BlockSpec((24, 10), lambda i: (0, 0)),
          pl.BlockSpec((1, 10), lambda i: (0, 0)),
      ],
      out_specs=[
          pl.BlockSpec((24, _K5_BLK), lambda i: (0, i)),
          pl.BlockSpec((10, _K5_BLK), lambda i: (0, i)),
      ],
      out_shape=[
          jax.ShapeDtypeStruct((24, _E), _f32),
          jax.ShapeDtypeStruct((10, _E), _f32),
      ],
  )(ee, CW, Cb.reshape(1, 10))

  return (ot.T, et.T)


# speedup vs baseline: 18.1819x; 1.0691x over previous
import functools
import jax
import jax.numpy as jnp
from jax import lax
from jax.experimental import pallas as pl
from jax.experimental.pallas import tpu as pltpu
from jax.experimental.pallas import tpu_sc as plsc

_N = 10000
_E = 320000

_NC = 2
_NS = 16
_NW = _NC * _NS

_NPAD = 10112
_RPT = _NPAD // _NS
_W = 32
_H = 16
_CH = 128
_PCH = 160
_NBUF = 4
_EPAD = _NS * _PCH * _CH
_EPT = _E // _NW
_FCH = _EPT // _CH
_FTAIL = _EPT - _FCH * _CH

_mesh = plsc.VectorSubcoreMesh(
    core_axis_name="c", subcore_axis_name="s",
    num_cores=_NC, num_subcores=_NS)

_sc_params = pltpu.CompilerParams(use_tc_tiling_on_sc=False,
                                  needs_layout_passes=False)

_f32 = jnp.float32


def _rsqrt16(v):
  x2 = v * 0.5
  i = plsc.bitcast(v, jnp.int32)
  i = jnp.int32(0x5F3759DF) - lax.shift_right_arithmetic(i, 1)
  y = plsc.bitcast(i, _f32)
  for _ in range(3):
    y = y * (1.5 - x2 * y * y)
  return y


def _tanh16(z):
  return 1.0 - 2.0 / (jnp.exp(2.0 * z) + 1.0)


def _ring_prop(table, src_v, dst_v, acc_sh, rows, gsem, ssem):
  for b in range(_NBUF):
    pltpu.async_copy(table.at[src_v.at[b]], rows[b], gsem[b])

  nround = _PCH // _NBUF

  def round_(k, carry):
    j0 = k * _NBUF
    for b in range(_NBUF):
      pltpu.make_async_copy(table.at[src_v.at[j0 + b]], rows[b],
                            gsem[b]).wait()
      pltpu.async_copy(rows[b], acc_sh.at[dst_v.at[j0 + b]], ssem[b],
                       add=True)
    for b in range(_NBUF):
      @pl.when(k < nround - 1)
      def _():
        pltpu.make_async_copy(rows[b], acc_sh.at[dst_v.at[j0 + b]],
                              ssem[b]).wait()
        pltpu.async_copy(table.at[src_v.at[j0 + _NBUF + b]], rows[b],
                         gsem[b])
    return carry

  lax.fori_loop(0, nround, round_, 0)
  for b in range(_NBUF):
    j = _PCH - _NBUF + b
    pltpu.make_async_copy(rows[b], acc_sh.at[dst_v.at[j]], ssem[b]).wait()


def _zero_acc(zeros_hbm, acc_sh, sid):
  pltpu.sync_copy(zeros_hbm.at[pl.ds(sid * _RPT, _RPT)],
                  acc_sh.at[pl.ds(sid * _RPT, _RPT)])


@functools.partial(
    pl.kernel,
    out_type=(jax.ShapeDtypeStruct((_NC, _NPAD, _H), _f32),
              jax.ShapeDtypeStruct((_NC, _NPAD, _H), _f32),
              jax.ShapeDtypeStruct((_NPAD, _H), _f32),
              jax.ShapeDtypeStruct((_NC, _NPAD, _H), _f32)),
    mesh=_mesh,
    scratch_types=[
        pltpu.VMEM((_PCH, _CH), jnp.int32),
        pltpu.VMEM((_PCH, _CH), jnp.int32),
        pltpu.VMEM((_CH, _H), _f32),
        pltpu.VMEM((_RPT, _H), _f32),
        pltpu.VMEM((_RPT, _H), _f32),
        pltpu.VMEM((_RPT, _H), _f32),
        pltpu.VMEM((1, _H), _f32),
        pltpu.VMEM_SHARED((_NPAD, _H), _f32),
        [pltpu.VMEM((_CH, _H), _f32)] * _NBUF,
        [pltpu.SemaphoreType.DMA] * _NBUF,
        [pltpu.SemaphoreType.DMA] * _NBUF,
        pltpu.SemaphoreType.DMA,
    ],
    compiler_params=_sc_params,
)
def _mega_kernel(h0h_hbm, srcp_hbm, dstp_hbm, ones_hbm, zeros_hbm, b1h_hbm,
                 s2_hbm, g2h_hbm, dinv_hbm, g1h_hbm,
                 src_v, dst_v, ones_v, dinv_v, wa, wb, b1_v,
                 acc_sh, rows, gsem, ssem, sem):
  cid = lax.axis_index("c")
  sid = lax.axis_index("s")
  rsl = pl.ds(sid * _RPT, _RPT)

  pltpu.sync_copy(srcp_hbm.at[sid], src_v)
  pltpu.sync_copy(dstp_hbm.at[sid], dst_v)
  pltpu.sync_copy(ones_hbm, ones_v)
  pltpu.sync_copy(b1h_hbm.at[cid], b1_v)
  _zero_acc(zeros_hbm, acc_sh, sid)
  plsc.subcore_barrier()

  def dfire(j, carry):
    pltpu.async_copy(ones_v, acc_sh.at[dst_v.at[j]], sem, add=True)
    return carry

  lax.fori_loop(0, _PCH, dfire, 0)

  def ddrain(j, carry):
    pltpu.make_async_copy(ones_v, acc_sh.at[dst_v.at[j]], sem).wait()
    return carry

  lax.fori_loop(0, _PCH, ddrain, 0)
  plsc.subcore_barrier()

  pltpu.sync_copy(acc_sh.at[rsl], dinv_v)

  def dinv_row(r, carry):
    dinv_v[r, :] = _rsqrt16(dinv_v[r, :] + 1.0)
    return carry

  lax.fori_loop(0, _RPT, dinv_row, 0)

  @pl.when(cid == 0)
  def _():
    pltpu.sync_copy(dinv_v, dinv_hbm.at[rsl])

  pltpu.sync_copy(h0h_hbm.at[cid, rsl], wa)

  def g1_row(r, carry):
    wa[r, :] = wa[r, :] * dinv_v[r, :]
    return carry

  lax.fori_loop(0, _RPT, g1_row, 0)
  pltpu.sync_copy(wa, g1h_hbm.at[cid, rsl])

  _zero_acc(zeros_hbm, acc_sh, sid)
  plsc.subcore_barrier()

  _ring_prop(g1h_hbm.at[cid], src_v, dst_v, acc_sh, rows, gsem, ssem)
  plsc.subcore_barrier()

  b1row = b1_v[0, :]
  pltpu.sync_copy(acc_sh.at[rsl], wb)

  def g2_row(r, carry):
    d = dinv_v[r, :]
    ph = d * (wb[r, :] + wa[r, :]) + b1row
    wb[r, :] = d * _tanh16(ph)
    return carry

  lax.fori_loop(0, _RPT, g2_row, 0)
  pltpu.sync_copy(wb, g2h_hbm.at[cid, rsl])

  _zero_acc(zeros_hbm, acc_sh, sid)
  plsc.subcore_barrier()

  _ring_prop(g2h_hbm.at[cid], src_v, dst_v, acc_sh, rows, gsem, ssem)
  plsc.subcore_barrier()

  pltpu.sync_copy(acc_sh.at[rsl], s2_hbm.at[cid, rsl])


@functools.partial(
    pl.kernel,
    out_type=jax.ShapeDtypeStruct((_NC, _NPAD, _H), _f32),
    mesh=_mesh,
    scratch_types=[
        pltpu.VMEM((_PCH, _CH), jnp.int32),
        pltpu.VMEM((_PCH, _CH), jnp.int32),
        pltpu.VMEM_SHARED((_NPAD, _H), _f32),
        [pltpu.VMEM((_CH, _H), _f32)] * _NBUF,
        [pltpu.SemaphoreType.DMA] * _NBUF,
        [pltpu.SemaphoreType.DMA] * _NBUF,
    ],
    compiler_params=_sc_params,
)
def _prop_kernel(gh_hbm, srcp_hbm, dstp_hbm, zeros_hbm, out_hbm,
                 src_v, dst_v, acc_sh, rows, gsem, ssem):
  cid = lax.axis_index("c")
  sid = lax.axis_index("s")
  rsl = pl.ds(sid * _RPT, _RPT)
  pltpu.sync_copy(srcp_hbm.at[sid], src_v)
  pltpu.sync_copy(dstp_hbm.at[sid], dst_v)
  _zero_acc(zeros_hbm, acc_sh, sid)
  plsc.subcore_barrier()
  _ring_prop(gh_hbm.at[cid], src_v, dst_v, acc_sh, rows, gsem, ssem)
  plsc.subcore_barrier()
  pltpu.sync_copy(acc_sh.at[rsl], out_hbm.at[cid, rsl])


@functools.partial(
    pl.kernel,
    out_type=jax.ShapeDtypeStruct((_E, 32), _f32),
    mesh=_mesh,
    scratch_types=[
        pltpu.VMEM((_EPT,), jnp.int32),
        pltpu.VMEM((_EPT,), jnp.int32),
        [pltpu.VMEM((_CH, 16), _f32)] * 2,
        [pltpu.VMEM((_CH, 16), _f32)] * 2,
        [pltpu.SemaphoreType.DMA] * 2,
        [pltpu.SemaphoreType.DMA] * 2,
    ],
    compiler_params=_sc_params,
)
def _edge_kernel(t_hbm, src_hbm, dst_hbm, ee_hbm,
                 src_v, dst_v, a_v, b_v, asem, bsem):
  wid = lax.axis_index("s") * _NC + lax.axis_index("c")
  base = wid * _EPT

  pltpu.sync_copy(src_hbm.at[pl.ds(base, _EPT)], src_v)
  pltpu.sync_copy(dst_hbm.at[pl.ds(base, _EPT)], dst_v)

  def chunk(j, b, nrows):
    row0 = base + j * _CH
    sidx = src_v.at[pl.ds(j * _CH, nrows)]
    didx = dst_v.at[pl.ds(j * _CH, nrows)]
    ga = pltpu.async_copy(t_hbm.at[sidx], a_v[b].at[pl.ds(0, nrows)],
                          asem[b])
    gb = pltpu.async_copy(t_hbm.at[didx], b_v[b].at[pl.ds(0, nrows)],
                          bsem[b])
    ga.wait()
    pltpu.sync_copy(a_v[b].at[pl.ds(0, nrows)],
                    ee_hbm.at[pl.ds(row0, nrows), pl.ds(0, 16)])
    gb.wait()
    pltpu.sync_copy(b_v[b].at[pl.ds(0, nrows)],
                    ee_hbm.at[pl.ds(row0, nrows), pl.ds(16, 16)])

  def body(j, carry):
    chunk(j, 0, _CH)
    return carry

  lax.fori_loop(0, _FCH, body, 0)
  chunk(_FCH, 1, _FTAIL)


_TCG = 8
_RBLK = _NPAD // _TCG


def _rblk(minor):
  return pl.BlockSpec((_RBLK, minor), lambda i: (i, 0))


def _rblk3(minor):
  return pl.BlockSpec((2, _RBLK, minor), lambda i: (0, i, 0))


def _full(shape):
  nd = len(shape)
  return pl.BlockSpec(shape, lambda i: (0,) * nd)


def _tc_call(body, out_shapes, in_specs, out_specs):
  return pl.pallas_call(body, grid=(_TCG,), in_specs=in_specs,
                        out_specs=out_specs, out_shape=out_shapes)


def _k1_body(x_ref, w1_ref, h0h_ref):
  h = jnp.dot(x_ref[...], w1_ref[...], preferred_element_type=_f32)
  h0h_ref[0] = h[:, 0:16]
  h0h_ref[1] = jnp.concatenate(
      [h[:, 16:18], jnp.zeros((_RBLK, 14), _f32)], axis=1)


def _k3_body(s_ref, g2h_ref, dinv_ref, w2_ref, b2_ref, g3h_ref):
  dinv = dinv_ref[:, 0:1]
  sg = s_ref[0] + g2h_ref[0]
  sg1 = s_ref[1] + g2h_ref[1]
  ph = (dinv * jnp.concatenate([sg, sg1[:, 0:2]], axis=1))
  h2 = jnp.tanh(
      jnp.dot(ph, w2_ref[...], preferred_element_type=_f32) + b2_ref[...])
  g3 = dinv * h2
  g3h_ref[0] = g3[:, 0:16]
  g3h_ref[1] = jnp.concatenate(
      [g3[:, 16:24], jnp.zeros((_RBLK, 8), _f32)], axis=1)


def _k4_body(s_ref, g3h_ref, dinv_ref, w3_ref, b3_ref, l1w_ref, l1b_ref,
             l2w_ref, l2b_ref, l3w_ref, l3b_ref, t_ref):
  dinv = dinv_ref[:, 0:1]
  sg = s_ref[0] + g3h_ref[0]
  sg1 = s_ref[1] + g3h_ref[1]
  ph = dinv * jnp.concatenate([sg, sg1[:, 0:8]], axis=1)
  t3 = jnp.tanh(
      jnp.dot(ph, w3_ref[...], preferred_element_type=_f32) + b3_ref[...])
  t4 = jnp.tanh(
      jnp.dot(t3, l1w_ref[...], preferred_element_type=_f32) + l1b_ref[...])
  t5 = jnp.tanh(
      jnp.dot(t4, l2w_ref[...], preferred_element_type=_f32) + l2b_ref[...])
  h = jnp.tanh(
      jnp.dot(t5, l3w_ref[...], preferred_element_type=_f32) + l3b_ref[...])
  t_ref[...] = jnp.concatenate([h, jnp.zeros((_RBLK, 4), _f32)], axis=1)


def _k5_body(ee_ref, cw_ref, cb_ref, et_ref, ot_ref):
  e = jnp.concatenate([ee_ref[:, 0:12], ee_ref[:, 16:28]], axis=1)
  et_ref[...] = e.T
  o = jnp.dot(e, cw_ref[...], preferred_element_type=_f32) + cb_ref[...]
  ot_ref[...] = o.T


_K5_BLK = 6400
_K5_R8 = _K5_BLK * 16 // 128


def kernel(x, edge_index, batch, W1, b1, W2, b2, W3, b3,
           L1W, L1b, L2W, L2b, L3W, L3b, CW, Cb):
  del batch
  src = edge_index[0].astype(jnp.int32)
  dst = edge_index[1].astype(jnp.int32)

  sent = jnp.full((_EPAD - _E,), _N, jnp.int32)
  srcp = jnp.concatenate([src, sent]).reshape(_NS, _PCH, _CH)
  dstp = jnp.concatenate([dst, sent]).reshape(_NS, _PCH, _CH)

  xpad = jnp.zeros((_NPAD, 128), _f32).at[:_N].set(x)
  ones16 = jnp.ones((_CH, _H), _f32)
  zeros16 = jnp.zeros((_NPAD, _H), _f32)
  b1h = jnp.zeros((_NC, 1, _H), _f32).at[0, 0, :16].set(b1[:16]) \
      .at[1, 0, :2].set(b1[16:18])

  h0h = _tc_call(
      _k1_body, jax.ShapeDtypeStruct((_NC, _NPAD, _H), _f32),
      [_rblk(128), _full((128, 18))], _rblk3(_H))(xpad, W1)

  s2, g2h, dinv16, _g1h = _mega_kernel(h0h, srcp, dstp, ones16, zeros16,
                                       b1h)

  g3h = _tc_call(
      _k3_body, jax.ShapeDtypeStruct((_NC, _NPAD, _H), _f32),
      [_rblk3(_H), _rblk3(_H), _rblk(_H), _full((18, 24)), _full((1, 24))],
      _rblk3(_H))(s2, g2h, dinv16, W2, b2.reshape(1, 24))

  s3 = _prop_kernel(g3h, srcp, dstp, zeros16)

  t = _tc_call(
      _k4_body, jax.ShapeDtypeStruct((_NPAD, 16), _f32),
      [_rblk3(_H), _rblk3(_H), _rblk(_H), _full((24, 256)), _full((1, 256)),
       _full((256, 24)), _full((1, 24)), _full((24, 18)), _full((1, 18)),
       _full((18, 12)), _full((1, 12))],
      _rblk(16))(s3, g3h, dinv16, W3, b3.reshape(1, 256), L1W,
                 L1b.reshape(1, 24), L2W, L2b.reshape(1, 18), L3W,
                 L3b.reshape(1, 12))

  ee = _edge_kernel(t, src, dst)

  et, ot = pl.pallas_call(
      _k5_body,
      grid=(_E // _K5_BLK,),
      in_specs=[
          pl.BlockSpec((_K5_BLK, 32), lambda i: (i, 0)),
          pl.BlockSpec((24, 10), lambda i: (0, 0)),
          pl.BlockSpec((1, 10), lambda i: (0, 0)),
      ],
      out_specs=[
          pl.BlockSpec((24, _K5_BLK), lambda i: (0, i)),
          pl.BlockSpec((10, _K5_BLK), lambda i: (0, i)),
      ],
      out_shape=[
          jax.ShapeDtypeStruct((24, _E), _f32),
          jax.ShapeDtypeStruct((10, _E), _f32),
      ],
  )(ee, CW, Cb.reshape(1, 10))

  return (ot.T, et.T)
